# baseline (speedup 1.0000x reference)
import jax
import jax.numpy as jnp
from jax.experimental import pallas as pl
from jax.experimental.pallas import tpu as pltpu

_NUM_CODES = 512
_BETA = 0.25


def _vq_body(x_ref, emb2_ref, embT_hi_ref, embT_lo_ref, emb_ref,
             zq_ref, codes_ref, loss_ref):
    nb, C, H, W = x_ref.shape
    HW = H * W
    x = jnp.concatenate(
        [x_ref[i].reshape(C, HW) for i in range(nb)], axis=1) \
        if nb > 1 else x_ref[0].reshape(C, HW)
    emb2 = emb2_ref[...]
    emb = emb_ref[...]

    xsq = jnp.sum(x * x, axis=0)
    esq = jnp.sum(emb * emb, axis=1)
    m2 = jax.lax.dot_general(
        emb2, x, (((1,), (0,)), ((), ())),
        preferred_element_type=jnp.float32)
    dist = (esq[:, None] + xsq[None, :]) - m2

    mn = jnp.min(dist, axis=0, keepdims=True)
    code_iota = jax.lax.broadcasted_iota(jnp.int32, dist.shape, 0)
    sel = jnp.where(dist == mn, code_iota, _NUM_CODES)
    codes = jnp.min(sel, axis=0)
    for i in range(nb):
        codes_ref[i, 0, :] = codes[i * HW:(i + 1) * HW]

    loss_ref[0, 0, 0] = jnp.sum(mn)

    onehot = (code_iota == codes[None, :]).astype(jnp.bfloat16)
    zqT = (jax.lax.dot_general(
               embT_hi_ref[...], onehot, (((1,), (0,)), ((), ())),
               preferred_element_type=jnp.float32)
           + jax.lax.dot_general(
               embT_lo_ref[...], onehot, (((1,), (0,)), ((), ())),
               preferred_element_type=jnp.float32))

    zq_st = x + (zqT - x)
    for i in range(nb):
        zq_ref[i] = zq_st[:, i * HW:(i + 1) * HW].reshape(C, H, W)


def kernel(z_e, emb):
    B, C, H, W = z_e.shape
    NB = 2

    embT = emb.T
    embT_hi = embT.astype(jnp.bfloat16)
    embT_lo = (embT - embT_hi.astype(jnp.float32)).astype(jnp.bfloat16)

    zq_st, codes3, lossp = pl.pallas_call(
        _vq_body,
        grid=(B // NB,),
        in_specs=[
            pl.BlockSpec((NB, C, H, W), lambda m: (m, 0, 0, 0)),
            pl.BlockSpec((_NUM_CODES, C), lambda m: (0, 0)),
            pl.BlockSpec((C, _NUM_CODES), lambda m: (0, 0)),
            pl.BlockSpec((C, _NUM_CODES), lambda m: (0, 0)),
            pl.BlockSpec((_NUM_CODES, C), lambda m: (0, 0)),
        ],
        out_specs=[
            pl.BlockSpec((NB, C, H, W), lambda m: (m, 0, 0, 0)),
            pl.BlockSpec((NB, 1, H * W), lambda m: (m, 0, 0)),
            pl.BlockSpec((1, 1, 1), lambda m: (m, 0, 0),
                         memory_space=pltpu.SMEM),
        ],
        out_shape=[
            jax.ShapeDtypeStruct((B, C, H, W), jnp.float32),
            jax.ShapeDtypeStruct((B, 1, H * W), jnp.int32),
            jax.ShapeDtypeStruct((B // NB, 1, 1), jnp.float32),
        ],
    )(z_e, emb * 2.0, embT_hi, embT_lo, emb)

    codes = codes3.reshape(B, H, W)
    vq_loss = (1.0 + _BETA) * jnp.sum(lossp) / (B * C * H * W)
    return zq_st, vq_loss, codes


# speedup vs baseline: 1.0348x; 1.0348x over previous
import jax
import jax.numpy as jnp
from jax.experimental import pallas as pl
from jax.experimental.pallas import tpu as pltpu

_NUM_CODES = 512
_BETA = 0.25


def _vq_body(x_ref, emb2_ref, embT_hi_ref, embT_lo_ref, emb_ref,
             zq_ref, codes_ref, loss_ref):
    R, HW = x_ref.shape
    C = 64
    nb = R // C
    x = jnp.concatenate(
        [x_ref[i * C:(i + 1) * C] for i in range(nb)], axis=1) \
        if nb > 1 else x_ref[...]
    emb2 = emb2_ref[...]
    emb = emb_ref[...]

    xsq = jnp.sum(x * x, axis=0)
    esq = jnp.sum(emb * emb, axis=1)
    m2 = jax.lax.dot_general(
        emb2, x, (((1,), (0,)), ((), ())),
        preferred_element_type=jnp.float32)
    dist = (esq[:, None] + xsq[None, :]) - m2

    mn = jnp.min(dist, axis=0, keepdims=True)
    code_iota = jax.lax.broadcasted_iota(jnp.int32, dist.shape, 0)
    sel = jnp.where(dist == mn, code_iota, _NUM_CODES)
    codes = jnp.min(sel, axis=0)
    for i in range(nb):
        codes_ref[i, 0, :] = codes[i * HW:(i + 1) * HW]

    loss_ref[0, 0, 0] = jnp.sum(mn)

    onehot = (code_iota == codes[None, :]).astype(jnp.bfloat16)
    zqT = (jax.lax.dot_general(
               embT_hi_ref[...], onehot, (((1,), (0,)), ((), ())),
               preferred_element_type=jnp.float32)
           + jax.lax.dot_general(
               embT_lo_ref[...], onehot, (((1,), (0,)), ((), ())),
               preferred_element_type=jnp.float32))

    zq_st = x + (zqT - x)
    for i in range(nb):
        zq_ref[i * C:(i + 1) * C] = zq_st[:, i * HW:(i + 1) * HW]


def kernel(z_e, emb):
    B, C, H, W = z_e.shape
    HW = H * W
    NB = 2
    z2 = z_e.reshape(B * C, HW)

    embT = emb.T
    embT_hi = embT.astype(jnp.bfloat16)
    embT_lo = (embT - embT_hi.astype(jnp.float32)).astype(jnp.bfloat16)

    zq_st, codes2, lossp = pl.pallas_call(
        _vq_body,
        grid=(B // NB,),
        in_specs=[
            pl.BlockSpec((NB * C, HW), lambda m: (m, 0)),
            pl.BlockSpec((_NUM_CODES, C), lambda m: (0, 0)),
            pl.BlockSpec((C, _NUM_CODES), lambda m: (0, 0)),
            pl.BlockSpec((C, _NUM_CODES), lambda m: (0, 0)),
            pl.BlockSpec((_NUM_CODES, C), lambda m: (0, 0)),
        ],
        out_specs=[
            pl.BlockSpec((NB * C, HW), lambda m: (m, 0)),
            pl.BlockSpec((NB, 1, HW), lambda m: (m, 0, 0)),
            pl.BlockSpec((1, 1, 1), lambda m: (m, 0, 0),
                         memory_space=pltpu.SMEM),
        ],
        out_shape=[
            jax.ShapeDtypeStruct((B * C, HW), jnp.float32),
            jax.ShapeDtypeStruct((B, 1, HW), jnp.int32),
            jax.ShapeDtypeStruct((B // NB, 1, 1), jnp.float32),
        ],
    )(z2, emb * 2.0, embT_hi, embT_lo, emb)

    zq_st = zq_st.reshape(B, C, H, W)
    codes = codes2.reshape(B, H, W)
    vq_loss = (1.0 + _BETA) * jnp.sum(lossp) / (B * C * H * W)
    return zq_st, vq_loss, codes


# speedup vs baseline: 1.7682x; 1.7088x over previous
import jax
import jax.numpy as jnp
from jax.experimental import pallas as pl
from jax.experimental.pallas import tpu as pltpu

_NUM_CODES = 512
_BETA = 0.25


def _vq_body(x_ref, emb2_ref, embT_hi_ref, embT_lo_ref, emb_ref,
             zq_ref, codes_ref, loss_ref):
    nb = x_ref.shape[0]
    x = jnp.concatenate([x_ref[i] for i in range(nb)], axis=1) \
        if nb > 1 else x_ref[0]
    emb2 = emb2_ref[...]
    emb = emb_ref[...]

    xsq = jnp.sum(x * x, axis=0)
    esq = jnp.sum(emb * emb, axis=1)
    m2 = jax.lax.dot_general(
        emb2, x, (((1,), (0,)), ((), ())),
        preferred_element_type=jnp.float32)
    dist = (esq[:, None] + xsq[None, :]) - m2

    mn = jnp.min(dist, axis=0, keepdims=True)
    code_iota = jax.lax.broadcasted_iota(
        jnp.int32, (_NUM_CODES, 1), 0).astype(jnp.float32)
    sel = jnp.where(dist == mn, code_iota, float(_NUM_CODES))
    codes_f = jnp.min(sel, axis=0)
    codes = codes_f.astype(jnp.int32)
    HW = codes.shape[0] // nb
    for i in range(nb):
        codes_ref[i, 0, :] = codes[i * HW:(i + 1) * HW]

    loss_ref[0, 0, 0] = jnp.sum(mn)

    onehot = (code_iota == codes_f[None, :]).astype(jnp.bfloat16)
    zqT = (jax.lax.dot_general(
               embT_hi_ref[...], onehot, (((1,), (0,)), ((), ())),
               preferred_element_type=jnp.float32)
           + jax.lax.dot_general(
               embT_lo_ref[...], onehot, (((1,), (0,)), ((), ())),
               preferred_element_type=jnp.float32))

    zq_st = x + (zqT - x)
    for i in range(nb):
        zq_ref[i] = zq_st[:, i * HW:(i + 1) * HW]


def kernel(z_e, emb):
    B, C, H, W = z_e.shape
    HW = H * W
    NB = 2
    z3 = z_e.reshape(B, C, HW)

    embT = emb.T
    embT_hi = embT.astype(jnp.bfloat16)
    embT_lo = (embT - embT_hi.astype(jnp.float32)).astype(jnp.bfloat16)

    zq_st, codes3, lossp = pl.pallas_call(
        _vq_body,
        grid=(B // NB,),
        in_specs=[
            pl.BlockSpec((NB, C, HW), lambda m: (m, 0, 0)),
            pl.BlockSpec((_NUM_CODES, C), lambda m: (0, 0)),
            pl.BlockSpec((C, _NUM_CODES), lambda m: (0, 0)),
            pl.BlockSpec((C, _NUM_CODES), lambda m: (0, 0)),
            pl.BlockSpec((_NUM_CODES, C), lambda m: (0, 0)),
        ],
        out_specs=[
            pl.BlockSpec((NB, C, HW), lambda m: (m, 0, 0)),
            pl.BlockSpec((NB, 1, HW), lambda m: (m, 0, 0)),
            pl.BlockSpec((1, 1, 1), lambda m: (m, 0, 0),
                         memory_space=pltpu.SMEM),
        ],
        out_shape=[
            jax.ShapeDtypeStruct((B, C, HW), jnp.float32),
            jax.ShapeDtypeStruct((B, 1, HW), jnp.int32),
            jax.ShapeDtypeStruct((B // NB, 1, 1), jnp.float32),
        ],
    )(z3, emb * 2.0, embT_hi, embT_lo, emb)

    zq_st = zq_st.reshape(B, C, H, W)
    codes = codes3.reshape(B, H, W)
    vq_loss = (1.0 + _BETA) * jnp.sum(lossp) / (B * C * H * W)
    return zq_st, vq_loss, codes


# speedup vs baseline: 1.7709x; 1.0016x over previous
import jax
import jax.numpy as jnp
from jax.experimental import pallas as pl
from jax.experimental.pallas import tpu as pltpu

_NUM_CODES = 512
_BETA = 0.25


def _vq_body(x_ref, emb_ref, zq_ref, codes_ref, loss_ref,
             emb2_s, hi_s, lo_s):
    nb = x_ref.shape[0]
    x = jnp.concatenate([x_ref[i] for i in range(nb)], axis=1) \
        if nb > 1 else x_ref[0]
    emb = emb_ref[...]

    @pl.when(pl.program_id(0) == 0)
    def _prep():
        emb2_s[...] = emb + emb
        embT = emb.T
        hi = embT.astype(jnp.bfloat16)
        hi_s[...] = hi
        lo_s[...] = (embT - hi.astype(jnp.float32)).astype(jnp.bfloat16)

    xsq = jnp.sum(x * x, axis=0)
    esq = jnp.sum(emb * emb, axis=1)
    m2 = jax.lax.dot_general(
        emb2_s[...], x, (((1,), (0,)), ((), ())),
        preferred_element_type=jnp.float32)
    dist = (esq[:, None] + xsq[None, :]) - m2

    mn = jnp.min(dist, axis=0, keepdims=True)
    code_iota = jax.lax.broadcasted_iota(
        jnp.int32, (_NUM_CODES, 1), 0).astype(jnp.float32)
    sel = jnp.where(dist == mn, code_iota, float(_NUM_CODES))
    codes_f = jnp.min(sel, axis=0)
    codes = codes_f.astype(jnp.int32)
    HW = codes.shape[0] // nb
    for i in range(nb):
        codes_ref[i, 0, :] = codes[i * HW:(i + 1) * HW]

    loss_ref[0, 0, 0] = jnp.sum(mn)

    onehot = (code_iota == codes_f[None, :]).astype(jnp.bfloat16)
    zqT = (jax.lax.dot_general(
               hi_s[...], onehot, (((1,), (0,)), ((), ())),
               preferred_element_type=jnp.float32)
           + jax.lax.dot_general(
               lo_s[...], onehot, (((1,), (0,)), ((), ())),
               preferred_element_type=jnp.float32))

    zq_st = x + (zqT - x)
    for i in range(nb):
        zq_ref[i] = zq_st[:, i * HW:(i + 1) * HW]


def kernel(z_e, emb):
    B, C, H, W = z_e.shape
    HW = H * W
    NB = 2
    z3 = z_e.reshape(B, C, HW)

    zq_st, codes3, lossp = pl.pallas_call(
        _vq_body,
        grid=(B // NB,),
        in_specs=[
            pl.BlockSpec((NB, C, HW), lambda m: (m, 0, 0)),
            pl.BlockSpec((_NUM_CODES, C), lambda m: (0, 0)),
        ],
        out_specs=[
            pl.BlockSpec((NB, C, HW), lambda m: (m, 0, 0)),
            pl.BlockSpec((NB, 1, HW), lambda m: (m, 0, 0)),
            pl.BlockSpec((1, 1, 1), lambda m: (m, 0, 0),
                         memory_space=pltpu.SMEM),
        ],
        out_shape=[
            jax.ShapeDtypeStruct((B, C, HW), jnp.float32),
            jax.ShapeDtypeStruct((B, 1, HW), jnp.int32),
            jax.ShapeDtypeStruct((B // NB, 1, 1), jnp.float32),
        ],
        scratch_shapes=[
            pltpu.VMEM((_NUM_CODES, C), jnp.float32),
            pltpu.VMEM((C, _NUM_CODES), jnp.bfloat16),
            pltpu.VMEM((C, _NUM_CODES), jnp.bfloat16),
        ],
    )(z3, emb)

    zq_st = zq_st.reshape(B, C, H, W)
    codes = codes3.reshape(B, H, W)
    vq_loss = (1.0 + _BETA) * jnp.sum(lossp) / (B * C * H * W)
    return zq_st, vq_loss, codes
